# Initial kernel scaffold; baseline (speedup 1.0000x reference)
import jax
import jax.numpy as jnp
from jax.experimental import pallas as pl


def kernel(z, codebooks):
    raise NotImplementedError("write your pallas kernel here")


# speedup vs baseline: 9.9436x; 9.9436x over previous
import jax
import jax.numpy as jnp
from jax.experimental import pallas as pl
from jax.experimental.pallas import tpu as pltpu


def _vq_tile_kernel(z_ref, cb_ref, zq_ref, idx_ref, loss_ref):
    z_blk = z_ref[0, 0]
    cb = cb_ref[0]

    c2 = jnp.sum(cb * cb, axis=1, keepdims=True)
    z2 = jnp.sum(z_blk * z_blk, axis=0, keepdims=True)
    ncross2 = jax.lax.dot_general(
        cb * -2.0, z_blk, (((1,), (0,)), ((), ())),
        preferred_element_type=jnp.float32)
    d2p = (z2 + ncross2) + c2

    idx = jnp.argmin(d2p, axis=0, keepdims=True)

    onehot = (jax.lax.broadcasted_iota(jnp.int32, d2p.shape, 0) == idx
              ).astype(jnp.float32)
    z_q = jax.lax.dot_general(
        cb, onehot, (((0,), (0,)), ((), ())),
        preferred_element_type=jnp.float32)

    diff = z_q - z_blk
    zq_ref[0, 0] = z_blk + diff
    idx_ref[0, 0] = idx.astype(jnp.int32)
    loss_ref[0, 0, 0, 0] = jnp.sum(diff * diff)


def kernel(z, codebooks):
    B, D, H, W = z.shape
    N, M, ds = codebooks.shape
    HW = H * W
    T = B * HW

    zr = z.reshape(B, N, ds, HW)

    grid = (B, N)
    zq4, idx4, loss_p = pl.pallas_call(
        _vq_tile_kernel,
        grid=grid,
        in_specs=[
            pl.BlockSpec((1, 1, ds, HW), lambda b, n: (b, n, 0, 0)),
            pl.BlockSpec((1, M, ds), lambda b, n: (n, 0, 0)),
        ],
        out_specs=[
            pl.BlockSpec((1, 1, ds, HW), lambda b, n: (b, n, 0, 0)),
            pl.BlockSpec((1, 1, 1, HW), lambda b, n: (b, n, 0, 0)),
            pl.BlockSpec((1, 1, 1, 1), lambda b, n: (b, n, 0, 0),
                         memory_space=pltpu.SMEM),
        ],
        out_shape=[
            jax.ShapeDtypeStruct((B, N, ds, HW), jnp.float32),
            jax.ShapeDtypeStruct((B, N, 1, HW), jnp.int32),
            jax.ShapeDtypeStruct((B, N, 1, 1), jnp.float32),
        ],
        compiler_params=pltpu.CompilerParams(
            dimension_semantics=("parallel", "parallel")),
    )(zr, codebooks)

    z_q_out = zq4.reshape(B, D, H, W)
    loss = jnp.sum(loss_p) / jnp.float32(N * T * ds)
    indices = idx4.reshape(B, N, HW).transpose(0, 2, 1).reshape(T, N)
    return (z_q_out, loss, loss, indices)


# speedup vs baseline: 11.4011x; 1.1466x over previous
import jax
import jax.numpy as jnp
from jax.experimental import pallas as pl
from jax.experimental.pallas import tpu as pltpu


def _vq_batch_kernel(z_ref, cb_ref, zq_ref, idx_ref, loss_ref):
    n_sub = cb_ref.shape[0]
    idx_rows = []
    loss = None
    for n in range(n_sub):
        z_blk = z_ref[0, n]
        cb = cb_ref[n]

        c2 = jnp.sum(cb * cb, axis=1, keepdims=True)
        z2 = jnp.sum(z_blk * z_blk, axis=0, keepdims=True)
        ncross2 = jax.lax.dot_general(
            cb * -2.0, z_blk, (((1,), (0,)), ((), ())),
            preferred_element_type=jnp.float32)
        d2 = (z2 + ncross2) + c2

        idx = jnp.argmin(d2, axis=0, keepdims=True)

        onehot = (jax.lax.broadcasted_iota(jnp.int32, d2.shape, 0) == idx
                  ).astype(jnp.float32)
        z_q = jax.lax.dot_general(
            cb, onehot, (((0,), (0,)), ((), ())),
            preferred_element_type=jnp.float32)

        diff = z_q - z_blk
        zq_ref[0, n] = z_blk + diff
        part = jnp.sum(diff * diff)
        loss = part if loss is None else loss + part
        idx_rows.append(idx.astype(jnp.int32))

    idx_mat = jnp.concatenate(idx_rows, axis=0)
    idx_ref[0] = idx_mat.T
    loss_ref[0, 0, 0] = loss


def kernel(z, codebooks):
    B, D, H, W = z.shape
    N, M, ds = codebooks.shape
    HW = H * W
    T = B * HW

    zr = z.reshape(B, N, ds, HW)

    zq4, idx3, loss_p = pl.pallas_call(
        _vq_batch_kernel,
        grid=(B,),
        in_specs=[
            pl.BlockSpec((1, N, ds, HW), lambda b: (b, 0, 0, 0)),
            pl.BlockSpec((N, M, ds), lambda b: (0, 0, 0)),
        ],
        out_specs=[
            pl.BlockSpec((1, N, ds, HW), lambda b: (b, 0, 0, 0)),
            pl.BlockSpec((1, HW, N), lambda b: (b, 0, 0)),
            pl.BlockSpec((1, 1, 1), lambda b: (b, 0, 0),
                         memory_space=pltpu.SMEM),
        ],
        out_shape=[
            jax.ShapeDtypeStruct((B, N, ds, HW), jnp.float32),
            jax.ShapeDtypeStruct((B, HW, N), jnp.int32),
            jax.ShapeDtypeStruct((B, 1, 1), jnp.float32),
        ],
        compiler_params=pltpu.CompilerParams(
            dimension_semantics=("parallel",)),
    )(zr, codebooks)

    z_q_out = zq4.reshape(B, D, H, W)
    loss = jnp.sum(loss_p) / jnp.float32(N * T * ds)
    indices = idx3.reshape(T, N)
    return (z_q_out, loss, loss, indices)


# speedup vs baseline: 14.9392x; 1.3103x over previous
import jax
import jax.numpy as jnp
from jax.experimental import pallas as pl
from jax.experimental.pallas import tpu as pltpu


def _vq_batch_kernel(z_ref, cb_ref, zq_ref, idx_ref, loss_ref):
    n_sub = cb_ref.shape[0]
    ds = cb_ref.shape[2]
    idx_rows = []
    loss = None
    for n in range(n_sub):
        z_blk = z_ref[0, n * ds:(n + 1) * ds]
        cb = cb_ref[n]

        c2 = jnp.sum(cb * cb, axis=1, keepdims=True)
        z2 = jnp.sum(z_blk * z_blk, axis=0, keepdims=True)
        ncross2 = jax.lax.dot_general(
            cb * -2.0, z_blk, (((1,), (0,)), ((), ())),
            preferred_element_type=jnp.float32)
        d2 = (z2 + ncross2) + c2

        idx = jnp.argmin(d2, axis=0, keepdims=True)

        onehot = (jax.lax.broadcasted_iota(jnp.int32, d2.shape, 0) == idx
                  ).astype(jnp.float32)
        z_q = jax.lax.dot_general(
            cb, onehot, (((0,), (0,)), ((), ())),
            preferred_element_type=jnp.float32)

        diff = z_q - z_blk
        zq_ref[0, n * ds:(n + 1) * ds] = z_blk + diff
        part = jnp.sum(diff * diff)
        loss = part if loss is None else loss + part
        idx_rows.append(idx.astype(jnp.int32))

    idx_mat = jnp.concatenate(idx_rows, axis=0)
    idx_ref[0] = idx_mat.T
    loss_ref[0, 0, 0] = loss


def kernel(z, codebooks):
    B, D, H, W = z.shape
    N, M, ds = codebooks.shape
    HW = H * W
    T = B * HW

    zr = z.reshape(B, D, HW)

    zq3, idx3, loss_p = pl.pallas_call(
        _vq_batch_kernel,
        grid=(B,),
        in_specs=[
            pl.BlockSpec((1, D, HW), lambda b: (b, 0, 0)),
            pl.BlockSpec((N, M, ds), lambda b: (0, 0, 0)),
        ],
        out_specs=[
            pl.BlockSpec((1, D, HW), lambda b: (b, 0, 0)),
            pl.BlockSpec((1, HW, N), lambda b: (b, 0, 0)),
            pl.BlockSpec((1, 1, 1), lambda b: (b, 0, 0),
                         memory_space=pltpu.SMEM),
        ],
        out_shape=[
            jax.ShapeDtypeStruct((B, D, HW), jnp.float32),
            jax.ShapeDtypeStruct((B, HW, N), jnp.int32),
            jax.ShapeDtypeStruct((B, 1, 1), jnp.float32),
        ],
        compiler_params=pltpu.CompilerParams(
            dimension_semantics=("parallel",)),
    )(zr, codebooks)

    z_q_out = zq3.reshape(B, D, H, W)
    loss = jnp.sum(loss_p) / jnp.float32(N * T * ds)
    indices = idx3.reshape(T, N)
    return (z_q_out, loss, loss, indices)


# speedup vs baseline: 16.8518x; 1.1280x over previous
import jax
import jax.numpy as jnp
from jax.experimental import pallas as pl
from jax.experimental.pallas import tpu as pltpu


def _vq_batch_kernel(z_ref, cb_ref, zq_ref, idx_ref, loss_ref):
    n_sub = cb_ref.shape[0]
    ds = cb_ref.shape[2]
    idx_rows = []
    loss = None
    for n in range(n_sub):
        z_blk = z_ref[0, n * ds:(n + 1) * ds]
        cb = cb_ref[n]

        c2 = jnp.sum(cb * cb, axis=1, keepdims=True)
        ncross2 = jax.lax.dot_general(
            cb * -2.0, z_blk, (((1,), (0,)), ((), ())),
            preferred_element_type=jnp.float32)
        d2 = ncross2 + c2

        idx = jnp.argmin(d2, axis=0, keepdims=True)

        onehot = (jax.lax.broadcasted_iota(jnp.int32, d2.shape, 0) == idx
                  ).astype(jnp.float32)
        z_q = jax.lax.dot_general(
            cb, onehot, (((0,), (0,)), ((), ())),
            preferred_element_type=jnp.float32)

        diff = z_q - z_blk
        zq_ref[0, n * ds:(n + 1) * ds] = z_blk + diff
        part = jnp.sum(diff * diff)
        loss = part if loss is None else loss + part
        idx_rows.append(idx.astype(jnp.int32))

    idx_mat = jnp.concatenate(idx_rows, axis=0)
    idx_ref[0] = idx_mat.T
    loss_ref[0, 0, 0] = loss


def kernel(z, codebooks):
    B, D, H, W = z.shape
    N, M, ds = codebooks.shape
    HW = H * W
    T = B * HW

    zr = z.reshape(B, D, HW)

    zq3, idx3, loss_p = pl.pallas_call(
        _vq_batch_kernel,
        grid=(B,),
        in_specs=[
            pl.BlockSpec((1, D, HW), lambda b: (b, 0, 0)),
            pl.BlockSpec((N, M, ds), lambda b: (0, 0, 0)),
        ],
        out_specs=[
            pl.BlockSpec((1, D, HW), lambda b: (b, 0, 0)),
            pl.BlockSpec((1, HW, N), lambda b: (b, 0, 0)),
            pl.BlockSpec((1, 1, 1), lambda b: (b, 0, 0),
                         memory_space=pltpu.SMEM),
        ],
        out_shape=[
            jax.ShapeDtypeStruct((B, D, HW), jnp.float32),
            jax.ShapeDtypeStruct((B, HW, N), jnp.int32),
            jax.ShapeDtypeStruct((B, 1, 1), jnp.float32),
        ],
        compiler_params=pltpu.CompilerParams(
            dimension_semantics=("parallel",)),
    )(zr, codebooks)

    z_q_out = zq3.reshape(B, D, H, W)
    loss = jnp.sum(loss_p) / jnp.float32(N * T * ds)
    indices = idx3.reshape(T, N)
    return (z_q_out, loss, loss, indices)
